# initial kernel scaffold (unmeasured)
import jax
import jax.numpy as jnp
import numpy as np
from jax import lax
from jax.experimental import pallas as pl
from jax.experimental.pallas import tpu as pltpu

N_DEV = 4
SQ = 1024
D = 1024
HQ = 8
DH = 128
SCALE = 0.08838834764831843


def _rope_tables():
    inv = 1.0 / (10000.0 ** (np.arange(0, DH, 2) / DH))
    pos = np.arange(SQ)[:, None] * inv[None, :]
    cos = np.repeat(np.cos(pos), 2, axis=-1).astype(np.float32)
    sin = np.repeat(np.sin(pos), 2, axis=-1).astype(np.float32)
    P = np.zeros((DH, DH), np.float32)
    for k in range(DH // 2):
        P[2 * k + 1, 2 * k] = -1.0
        P[2 * k, 2 * k + 1] = 1.0
    return cos, sin, P


def _body(x_ref, wq_ref, wk_ref, wv_ref, wo_ref, cos_ref, sin_ref, p_ref,
          out_ref, xbuf, comm, ag_send, ag_recv, rs_send, rs_recv):
    i = lax.axis_index("i")
    right = lax.rem(i + 1, N_DEV)
    left = lax.rem(i + N_DEV - 1, N_DEV)

    barrier = pltpu.get_barrier_semaphore()
    for nbr in (left, right):
        pl.semaphore_signal(barrier, inc=1, device_id=(nbr,),
                            device_id_type=pl.DeviceIdType.MESH)
    pl.semaphore_wait(barrier, 2)

    xbuf[0] = x_ref[0]

    for h in range(N_DEV - 1):
        rdma = pltpu.make_async_remote_copy(
            src_ref=xbuf.at[h],
            dst_ref=xbuf.at[h + 1],
            send_sem=ag_send.at[h],
            recv_sem=ag_recv.at[h],
            device_id=(right,),
            device_id_type=pl.DeviceIdType.MESH,
        )
        rdma.start()
        rdma.wait()

    cos = cos_ref[...]
    sin = sin_ref[...]
    p = p_ref[...].astype(jnp.bfloat16)

    def compute_slot(s, _):
        xb = pl.load(xbuf, (pl.ds(s, 1), slice(None), slice(None)))[0]
        q = jnp.dot(xb, wq_ref[...], preferred_element_type=jnp.float32)
        k = jnp.dot(xb, wk_ref[...], preferred_element_type=jnp.float32)
        v = jnp.dot(xb, wv_ref[...], preferred_element_type=jnp.float32)
        v = v.astype(jnp.bfloat16)
        partial = jnp.zeros((SQ, D), jnp.float32)
        for h in range(HQ):
            sl = slice(h * DH, (h + 1) * DH)
            qh32, kh32 = q[:, sl], k[:, sl]
            q_rot = jnp.dot(qh32.astype(jnp.bfloat16), p,
                            preferred_element_type=jnp.float32)
            k_rot = jnp.dot(kh32.astype(jnp.bfloat16), p,
                            preferred_element_type=jnp.float32)
            qh = (qh32 * cos + q_rot * sin).astype(jnp.bfloat16)
            kh = (kh32 * cos + k_rot * sin).astype(jnp.bfloat16)
            s_mat = lax.dot_general(
                qh, kh, (((1,), (1,)), ((), ())),
                preferred_element_type=jnp.float32) * SCALE
            s_max = jnp.max(s_mat, axis=1, keepdims=True)
            w = jnp.exp(s_mat - s_max)
            w = (w / jnp.sum(w, axis=1, keepdims=True)).astype(jnp.bfloat16)
            ctx = jnp.dot(w, v[:, sl], preferred_element_type=jnp.float32)
            partial = partial + jnp.dot(ctx.astype(jnp.bfloat16),
                                        wo_ref[sl, :],
                                        preferred_element_type=jnp.float32)
        pl.store(xbuf, (pl.ds(s, 1), slice(None), slice(None)),
                 partial.astype(jnp.bfloat16)[None])
        return 0

    lax.fori_loop(0, N_DEV, compute_slot, 0)

    for h in range(N_DEV - 1):
        rdma = pltpu.make_async_remote_copy(
            src_ref=xbuf.at[h + 1],
            dst_ref=comm.at[h],
            send_sem=rs_send.at[h],
            recv_sem=rs_recv.at[h],
            device_id=(right,),
            device_id_type=pl.DeviceIdType.MESH,
        )
        rdma.start()
        rdma.wait()
        dst_slot = (h + 2) % N_DEV
        acc = (xbuf[dst_slot].astype(jnp.float32)
               + comm[h].astype(jnp.float32))
        if h < N_DEV - 2:
            xbuf[dst_slot] = acc.astype(jnp.bfloat16)
        else:
            out_ref[0, :, :] = acc


def kernel(x, Wq, Wk, Wv, Wo):
    cos_np, sin_np, p_np = _rope_tables()
    return pl.pallas_call(
        _body,
        out_shape=jax.ShapeDtypeStruct((1, SQ, D), jnp.float32),
        in_specs=[pl.BlockSpec(memory_space=pltpu.VMEM)] * 8,
        out_specs=pl.BlockSpec(memory_space=pltpu.VMEM),
        scratch_shapes=[
            pltpu.VMEM((N_DEV, SQ, D), jnp.bfloat16),
            pltpu.VMEM((N_DEV - 1, SQ, D), jnp.bfloat16),
            pltpu.SemaphoreType.DMA((N_DEV - 1,)),
            pltpu.SemaphoreType.DMA((N_DEV - 1,)),
            pltpu.SemaphoreType.DMA((N_DEV - 1,)),
            pltpu.SemaphoreType.DMA((N_DEV - 1,)),
        ],
        compiler_params=pltpu.CompilerParams(collective_id=0),
    )(
        x.astype(jnp.bfloat16),
        Wq.astype(jnp.bfloat16),
        Wk.astype(jnp.bfloat16),
        Wv.astype(jnp.bfloat16),
        Wo.astype(jnp.bfloat16),
        jnp.asarray(cos_np),
        jnp.asarray(sin_np),
        jnp.asarray(p_np),
    )


# baseline (device time: 278517 ns/iter reference)
import jax
import jax.numpy as jnp
import numpy as np
from jax import lax
from jax.experimental import pallas as pl
from jax.experimental.pallas import tpu as pltpu

N_DEV = 4
SQ = 1024
D = 1024
HQ = 8
DH = 128
SCALE = 0.08838834764831843


def _rope_tables():
    inv = 1.0 / (10000.0 ** (np.arange(0, DH, 2) / DH))
    pos = np.arange(SQ)[:, None] * inv[None, :]
    cos = np.repeat(np.cos(pos), 2, axis=-1).astype(np.float32)
    sin = np.repeat(np.sin(pos), 2, axis=-1).astype(np.float32)
    P = np.zeros((DH, DH), np.float32)
    for k in range(DH // 2):
        P[2 * k + 1, 2 * k] = -1.0
        P[2 * k, 2 * k + 1] = 1.0
    return cos, sin, P


def _body(x_ref, wq_ref, wk_ref, wv_ref, wo_ref, cos_ref, sin_ref, p_ref,
          out_ref, xbuf, comm, ag_send, ag_recv, rs_send, rs_recv):
    i = lax.axis_index("i")
    right = lax.rem(i + 1, N_DEV)
    left = lax.rem(i + N_DEV - 1, N_DEV)

    barrier = pltpu.get_barrier_semaphore()
    for nbr in (left, right):
        pl.semaphore_signal(barrier, inc=1, device_id=(nbr,),
                            device_id_type=pl.DeviceIdType.MESH)
    pl.semaphore_wait(barrier, 2)

    xbuf[0] = x_ref[0]

    for h in range(N_DEV - 1):
        rdma = pltpu.make_async_remote_copy(
            src_ref=xbuf.at[h],
            dst_ref=xbuf.at[h + 1],
            send_sem=ag_send.at[h],
            recv_sem=ag_recv.at[h],
            device_id=(right,),
            device_id_type=pl.DeviceIdType.MESH,
        )
        rdma.start()
        rdma.wait()

    cos = cos_ref[...]
    sin = sin_ref[...]
    p = p_ref[...].astype(jnp.bfloat16)

    def compute_slot(s, _):
        xb = xbuf[pl.ds(s, 1), :, :][0]
        q = jnp.dot(xb, wq_ref[...], preferred_element_type=jnp.float32)
        k = jnp.dot(xb, wk_ref[...], preferred_element_type=jnp.float32)
        v = jnp.dot(xb, wv_ref[...], preferred_element_type=jnp.float32)
        v = v.astype(jnp.bfloat16)
        partial = jnp.zeros((SQ, D), jnp.float32)
        for h in range(HQ):
            sl = slice(h * DH, (h + 1) * DH)
            qh32, kh32 = q[:, sl], k[:, sl]
            q_rot = jnp.dot(qh32.astype(jnp.bfloat16), p,
                            preferred_element_type=jnp.float32)
            k_rot = jnp.dot(kh32.astype(jnp.bfloat16), p,
                            preferred_element_type=jnp.float32)
            qh = (qh32 * cos + q_rot * sin).astype(jnp.bfloat16)
            kh = (kh32 * cos + k_rot * sin).astype(jnp.bfloat16)
            s_mat = lax.dot_general(
                qh, kh, (((1,), (1,)), ((), ())),
                preferred_element_type=jnp.float32) * SCALE
            s_max = jnp.max(s_mat, axis=1, keepdims=True)
            w = jnp.exp(s_mat - s_max)
            w = (w / jnp.sum(w, axis=1, keepdims=True)).astype(jnp.bfloat16)
            ctx = jnp.dot(w, v[:, sl], preferred_element_type=jnp.float32)
            partial = partial + jnp.dot(ctx.astype(jnp.bfloat16),
                                        wo_ref[sl, :],
                                        preferred_element_type=jnp.float32)
        xbuf[pl.ds(s, 1), :, :] = partial.astype(jnp.bfloat16)[None]
        return 0

    lax.fori_loop(0, N_DEV, compute_slot, 0)

    for h in range(N_DEV - 1):
        rdma = pltpu.make_async_remote_copy(
            src_ref=xbuf.at[h + 1],
            dst_ref=comm.at[h],
            send_sem=rs_send.at[h],
            recv_sem=rs_recv.at[h],
            device_id=(right,),
            device_id_type=pl.DeviceIdType.MESH,
        )
        rdma.start()
        rdma.wait()
        dst_slot = (h + 2) % N_DEV
        acc = (xbuf[dst_slot].astype(jnp.float32)
               + comm[h].astype(jnp.float32))
        if h < N_DEV - 2:
            xbuf[dst_slot] = acc.astype(jnp.bfloat16)
        else:
            out_ref[0, :, :] = acc


def kernel(x, Wq, Wk, Wv, Wo):
    cos_np, sin_np, p_np = _rope_tables()
    return pl.pallas_call(
        _body,
        out_shape=jax.ShapeDtypeStruct((1, SQ, D), jnp.float32),
        in_specs=[pl.BlockSpec(memory_space=pltpu.VMEM)] * 8,
        out_specs=pl.BlockSpec(memory_space=pltpu.VMEM),
        scratch_shapes=[
            pltpu.VMEM((N_DEV, SQ, D), jnp.bfloat16),
            pltpu.VMEM((N_DEV - 1, SQ, D), jnp.bfloat16),
            pltpu.SemaphoreType.DMA((N_DEV - 1,)),
            pltpu.SemaphoreType.DMA((N_DEV - 1,)),
            pltpu.SemaphoreType.DMA((N_DEV - 1,)),
            pltpu.SemaphoreType.DMA((N_DEV - 1,)),
        ],
        compiler_params=pltpu.CompilerParams(collective_id=0),
    )(
        x.astype(jnp.bfloat16),
        Wq.astype(jnp.bfloat16),
        Wk.astype(jnp.bfloat16),
        Wv.astype(jnp.bfloat16),
        Wo.astype(jnp.bfloat16),
        jnp.asarray(cos_np),
        jnp.asarray(sin_np),
        jnp.asarray(p_np),
    )


# device time: 177605 ns/iter; 1.5682x vs baseline; 1.5682x over previous
import jax
import jax.numpy as jnp
import numpy as np
from jax import lax
from jax.experimental import pallas as pl
from jax.experimental.pallas import tpu as pltpu

N_DEV = 4
SQ = 1024
D = 1024
HQ = 8
DH = 128
SCALE = 0.08838834764831843


def _rope_tables():
    inv = 1.0 / (10000.0 ** (np.arange(0, DH, 2) / DH))
    pos = np.arange(SQ)[:, None] * inv[None, :]
    cos = np.repeat(np.cos(pos), 2, axis=-1).astype(np.float32)
    sin = np.repeat(np.sin(pos), 2, axis=-1).astype(np.float32)
    P = np.zeros((DH, DH), np.float32)
    for k in range(DH // 2):
        P[2 * k + 1, 2 * k] = -1.0
        P[2 * k, 2 * k + 1] = 1.0
    return cos, sin, P


def _body(x_ref, wq_ref, wk_ref, wv_ref, wo_ref, cos_ref, sin_ref, p_ref,
          out_ref, xbuf, comm, ag_send, ag_recv, rs_send, rs_recv):
    i = lax.axis_index("i")
    right = lax.rem(i + 1, N_DEV)
    left = lax.rem(i + N_DEV - 1, N_DEV)

    barrier = pltpu.get_barrier_semaphore()
    for nbr in (left, right):
        pl.semaphore_signal(barrier, inc=1, device_id=(nbr,),
                            device_id_type=pl.DeviceIdType.MESH)
    pl.semaphore_wait(barrier, 2)

    ag = [
        pltpu.make_async_remote_copy(
            src_ref=xbuf.at[h],
            dst_ref=xbuf.at[h + 1],
            send_sem=ag_send.at[h],
            recv_sem=ag_recv.at[h],
            device_id=(right,),
            device_id_type=pl.DeviceIdType.MESH,
        )
        for h in range(N_DEV - 1)
    ]
    rs = [
        pltpu.make_async_remote_copy(
            src_ref=xbuf.at[h + 1],
            dst_ref=comm.at[h],
            send_sem=rs_send.at[h],
            recv_sem=rs_recv.at[h],
            device_id=(right,),
            device_id_type=pl.DeviceIdType.MESH,
        )
        for h in range(N_DEV - 1)
    ]

    cos = cos_ref[...]
    sin = sin_ref[...]
    p = p_ref[...].astype(jnp.bfloat16)

    def compute_slot(xb):
        q = jnp.dot(xb, wq_ref[...], preferred_element_type=jnp.float32)
        k = jnp.dot(xb, wk_ref[...], preferred_element_type=jnp.float32)
        v = jnp.dot(xb, wv_ref[...], preferred_element_type=jnp.float32)
        v = v.astype(jnp.bfloat16)
        partial = jnp.zeros((SQ, D), jnp.float32)
        for h in range(HQ):
            sl = slice(h * DH, (h + 1) * DH)
            qh32, kh32 = q[:, sl], k[:, sl]
            q_rot = jnp.dot(qh32.astype(jnp.bfloat16), p,
                            preferred_element_type=jnp.float32)
            k_rot = jnp.dot(kh32.astype(jnp.bfloat16), p,
                            preferred_element_type=jnp.float32)
            qh = (qh32 * cos + q_rot * sin).astype(jnp.bfloat16)
            kh = (kh32 * cos + k_rot * sin).astype(jnp.bfloat16)
            s_mat = lax.dot_general(
                qh, kh, (((1,), (1,)), ((), ())),
                preferred_element_type=jnp.float32) * SCALE
            s_max = jnp.max(s_mat, axis=1, keepdims=True)
            w = jnp.exp(s_mat - s_max)
            w = (w / jnp.sum(w, axis=1, keepdims=True)).astype(jnp.bfloat16)
            ctx = jnp.dot(w, v[:, sl], preferred_element_type=jnp.float32)
            partial = partial + jnp.dot(ctx.astype(jnp.bfloat16),
                                        wo_ref[sl, :],
                                        preferred_element_type=jnp.float32)
        return partial

    xbuf[0] = x_ref[0]
    ag[0].start()
    p0 = compute_slot(xbuf[0])
    ag[0].wait_send()
    xbuf[0] = p0.astype(jnp.bfloat16)

    ag[0].wait_recv()
    ag[1].start()
    p1 = compute_slot(xbuf[1])
    ag[1].wait_send()
    xbuf[1] = p1.astype(jnp.bfloat16)
    rs[0].start()

    ag[1].wait_recv()
    ag[2].start()
    p2 = compute_slot(xbuf[2])
    rs[0].wait_recv()
    acc2 = p2 + comm[0].astype(jnp.float32)
    ag[2].wait_send()
    xbuf[2] = acc2.astype(jnp.bfloat16)
    rs[1].start()

    ag[2].wait_recv()
    p3 = compute_slot(xbuf[3])
    rs[1].wait_recv()
    acc3 = p3 + comm[1].astype(jnp.float32)
    xbuf[3] = acc3.astype(jnp.bfloat16)
    rs[2].start()

    rs[2].wait_recv()
    out_ref[0, :, :] = (xbuf[0].astype(jnp.float32)
                        + comm[2].astype(jnp.float32))

    rs[0].wait_send()
    rs[1].wait_send()
    rs[2].wait_send()


def kernel(x, Wq, Wk, Wv, Wo):
    cos_np, sin_np, p_np = _rope_tables()
    return pl.pallas_call(
        _body,
        out_shape=jax.ShapeDtypeStruct((1, SQ, D), jnp.float32),
        in_specs=[pl.BlockSpec(memory_space=pltpu.VMEM)] * 8,
        out_specs=pl.BlockSpec(memory_space=pltpu.VMEM),
        scratch_shapes=[
            pltpu.VMEM((N_DEV, SQ, D), jnp.bfloat16),
            pltpu.VMEM((N_DEV - 1, SQ, D), jnp.bfloat16),
            pltpu.SemaphoreType.DMA((N_DEV - 1,)),
            pltpu.SemaphoreType.DMA((N_DEV - 1,)),
            pltpu.SemaphoreType.DMA((N_DEV - 1,)),
            pltpu.SemaphoreType.DMA((N_DEV - 1,)),
        ],
        compiler_params=pltpu.CompilerParams(collective_id=0),
    )(
        x.astype(jnp.bfloat16),
        Wq.astype(jnp.bfloat16),
        Wk.astype(jnp.bfloat16),
        Wv.astype(jnp.bfloat16),
        Wo.astype(jnp.bfloat16),
        jnp.asarray(cos_np),
        jnp.asarray(sin_np),
        jnp.asarray(p_np),
    )
